# baseline (speedup 1.0000x reference)
import functools

import jax
import jax.numpy as jnp
from jax import lax
from jax.experimental import pallas as pl
from jax.experimental.pallas import tpu as pltpu
from jax.experimental.pallas import tpu_sc as plsc

D = 128
L = 200
NC = 2
NS = 16
NW = NC * NS
NLANE = 16
NACC = D // NLANE
KP = 4
NPASS = L // KP


def _sc_body(text_r_hbm, table_hbm, out_hbm, idxs_v, acc_v, out_v, sem, isem):
    lpw = text_r_hbm.shape[1]
    bpw = lpw // L
    ch = KP * bpw
    wid = lax.axis_index("s") * NC + lax.axis_index("c")
    base = wid * bpw

    idx_copy = pltpu.async_copy(text_r_hbm.at[wid], idxs_v, isem)

    zeros = jnp.zeros((NLANE,), jnp.float32)

    def zero_body(i, carry):
        for c in range(NACC):
            acc_v[i, pl.ds(NLANE * c, NLANE)] = zeros
        return carry

    lax.fori_loop(0, KP * bpw, zero_body, 0)
    idx_copy.wait()

    def fire_body(t, carry):
        pltpu.async_copy(table_hbm.at[idxs_v.at[pl.ds(t * ch, ch)]], acc_v,
                         sem, add=True)
        return carry

    lax.fori_loop(0, NPASS, fire_body, 0)

    def drain_body(t, carry):
        pltpu.make_async_copy(table_hbm.at[pl.ds(0, ch)], acc_v, sem).wait()
        return carry

    lax.fori_loop(0, NPASS, drain_body, 0)

    scale = jnp.float32(1.0 / L)

    def scale_body(i, carry):
        for c in range(NACC):
            s = acc_v[i, pl.ds(NLANE * c, NLANE)]
            for p in range(1, KP):
                s = s + acc_v[p * bpw + i, pl.ds(NLANE * c, NLANE)]
            out_v[i, pl.ds(NLANE * c, NLANE)] = s * scale
        return carry

    lax.fori_loop(0, bpw, scale_body, 0)
    pltpu.sync_copy(out_v, out_hbm.at[pl.ds(base, bpw)])


def kernel(text, embeddings):
    batch = text.shape[0]
    bpw = batch // NW
    text_r = (text.astype(jnp.int32).T
              .reshape(L, NW, bpw).transpose(1, 0, 2).reshape(NW, L * bpw))
    run = functools.partial(
        pl.kernel,
        mesh=plsc.VectorSubcoreMesh(core_axis_name="c", subcore_axis_name="s"),
        out_type=jax.ShapeDtypeStruct((batch, D), jnp.float32),
        scratch_types=[
            pltpu.VMEM((L * bpw,), jnp.int32),
            pltpu.VMEM((KP * bpw, D), jnp.float32),
            pltpu.VMEM((bpw, D), jnp.float32),
            pltpu.SemaphoreType.DMA,
            pltpu.SemaphoreType.DMA,
        ],
    )(_sc_body)
    return run(text_r, embeddings)


# speedup vs baseline: 1.0188x; 1.0188x over previous
import functools

import jax
import jax.numpy as jnp
from jax import lax
from jax.experimental import pallas as pl
from jax.experimental.pallas import tpu as pltpu
from jax.experimental.pallas import tpu_sc as plsc

D = 128
L = 200
NC = 2
NS = 16
NW = NC * NS
NLANE = 16
NACC = D // NLANE


def _sc_body(text_t_hbm, table_hbm, out_hbm, idxs_v, acc_v, sem, isem):
    bpw = text_t_hbm.shape[1] // NW
    wid = lax.axis_index("s") * NC + lax.axis_index("c")
    base = wid * bpw

    idx_copy = pltpu.async_copy(text_t_hbm.at[:, pl.ds(base, bpw)], idxs_v,
                                isem)

    zeros = jnp.zeros((NLANE,), jnp.float32)

    def zero_body(i, carry):
        for c in range(NACC):
            acc_v[i, pl.ds(NLANE * c, NLANE)] = zeros
        return carry

    lax.fori_loop(0, bpw, zero_body, 0)
    idx_copy.wait()

    def fire_body(t, carry):
        pltpu.async_copy(table_hbm.at[idxs_v.at[t]], acc_v, sem, add=True)
        return carry

    lax.fori_loop(0, L, fire_body, 0)

    def drain_body(t, carry):
        pltpu.make_async_copy(table_hbm.at[pl.ds(0, bpw)], acc_v, sem).wait()
        return carry

    lax.fori_loop(0, L, drain_body, 0)

    scale = jnp.float32(1.0 / L)

    def scale_body(i, carry):
        for c in range(NACC):
            acc_v[i, pl.ds(NLANE * c, NLANE)] = (
                acc_v[i, pl.ds(NLANE * c, NLANE)] * scale)
        return carry

    lax.fori_loop(0, bpw, scale_body, 0)
    pltpu.sync_copy(acc_v, out_hbm.at[pl.ds(base, bpw)])


def kernel(text, embeddings):
    batch = text.shape[0]
    bpw = batch // NW
    run = functools.partial(
        pl.kernel,
        mesh=plsc.VectorSubcoreMesh(core_axis_name="c", subcore_axis_name="s"),
        out_type=jax.ShapeDtypeStruct((batch, D), jnp.float32),
        scratch_types=[
            pltpu.VMEM((L, bpw), jnp.int32),
            pltpu.VMEM((bpw, D), jnp.float32),
            pltpu.SemaphoreType.DMA,
            pltpu.SemaphoreType.DMA,
        ],
    )(_sc_body)
    return run(text.astype(jnp.int32).T, embeddings)
